# baseline (speedup 1.0000x reference)
import functools

import jax
import jax.numpy as jnp
from jax import lax
from jax.experimental import pallas as pl
from jax.experimental.pallas import tpu as pltpu
from jax.experimental.pallas import tpu_sc as plsc

_F = 26
_V = 1_000_000
_B = 16384
_NC, _NS, _L = 2, 16, 16
_NW = _NC * _NS
_BPW = _B // _NW

_mesh = plsc.VectorSubcoreMesh(
    core_axis_name="c", subcore_axis_name="s", num_cores=_NC, num_subcores=_NS
)


@functools.partial(
    pl.kernel,
    out_type=jax.ShapeDtypeStruct((_B,), jnp.float32),
    mesh=_mesh,
    scratch_types=[
        pltpu.VMEM((_F, _BPW), jnp.int32),
        pltpu.VMEM((_F, _BPW), jnp.float32),
        pltpu.VMEM((_BPW,), jnp.float32),
        pltpu.SemaphoreType.DMA,
    ],
    compiler_params=pltpu.CompilerParams(use_tc_tiling_on_sc=False),
)
def _emb_kernel(idx_hbm, table_hbm, out_hbm, idx_v, vals_v, out_v, sem):
    wid = lax.axis_index("s") * _NC + lax.axis_index("c")
    pltpu.sync_copy(idx_hbm.at[wid], idx_v)

    descs = [
        pltpu.async_copy(table_hbm.at[f].at[idx_v.at[f]], vals_v.at[f], sem)
        for f in range(_F)
    ]
    for d in descs:
        d.wait()

    for j in range(_BPW // _L):
        acc = None
        for f in range(_F):
            v = vals_v[f, pl.ds(j * _L, _L)]
            acc = v if acc is None else acc + v
        out_v[pl.ds(j * _L, _L)] = 1.0 / (1.0 + jnp.exp(-acc))

    pltpu.sync_copy(out_v, out_hbm.at[pl.ds(wid * _BPW, _BPW)])


def kernel(X, lin_table):
    idx = X.T.reshape(_F, _NW, _BPW).transpose(1, 0, 2)
    out = _emb_kernel(idx, lin_table)
    return out.reshape(_B, 1)


# speedup vs baseline: 5.9802x; 5.9802x over previous
import functools

import jax
import jax.numpy as jnp
from jax import lax
from jax.experimental import pallas as pl
from jax.experimental.pallas import tpu as pltpu
from jax.experimental.pallas import tpu_sc as plsc

_F = 26
_V = 1_000_000
_B = 16384
_NC, _NS, _L = 2, 16, 16
_NW = _NC * _NS
_BPW = _B // _NW
_IPW = _BPW * _F

_WIN = 2048
_NFULL = _V // _WIN
_REM = _V - _NFULL * _WIN
_REM_A = (_REM // 128) * 128
_REM_B = _REM - _REM_A

_mesh = plsc.VectorSubcoreMesh(
    core_axis_name="c", subcore_axis_name="s", num_cores=_NC, num_subcores=_NS
)


@functools.partial(
    pl.kernel,
    out_type=jax.ShapeDtypeStruct((_F * _V,), jnp.float32),
    mesh=_mesh,
    scratch_types=[
        pltpu.VMEM((_F, _WIN), jnp.float32),
        pltpu.VMEM((_WIN,), jnp.float32),
        pltpu.VMEM((_WIN,), jnp.float32),
        pltpu.SemaphoreType.DMA,
        pltpu.SemaphoreType.DMA,
    ],
)
def _detile_kernel(table_hbm, tail_hbm, lin_hbm, stripe_v, row0_v, row1_v,
                   isem, osem):
    wid = lax.axis_index("s") * _NC + lax.axis_index("c")
    rbufs = (row0_v, row1_v)

    def _do_stripe(off, width):
        pltpu.async_copy(
            table_hbm.at[:, pl.ds(off, width)],
            stripe_v.at[:, pl.ds(0, width)],
            isem,
        ).wait()
        descs = [None, None]
        for r in range(_F):
            rb = rbufs[r % 2]
            if descs[r % 2] is not None:
                descs[r % 2].wait()

            @pl.loop(0, width // _L)
            def _(c):
                rb[pl.ds(c * _L, _L)] = stripe_v[r, pl.ds(c * _L, _L)]

            descs[r % 2] = pltpu.async_copy(
                rb.at[pl.ds(0, width)],
                lin_hbm.at[pl.ds(r * _V + off, width)],
                osem,
            )
        for d in descs:
            if d is not None:
                d.wait()

    @pl.loop(0, (_NFULL + _NW - 1) // _NW)
    def _(k):
        w = k * _NW + wid

        @pl.when(w < _NFULL)
        def _():
            off = pl.multiple_of(w * _WIN, _WIN)
            _do_stripe(off, _WIN)

    @pl.when(wid == 0)
    def _():
        _do_stripe(_NFULL * _WIN, _REM_A)

    @pl.when(wid == 1)
    def _():
        pltpu.sync_copy(tail_hbm, row0_v.at[pl.ds(0, _F * _REM_B)])
        for r in range(_F):
            pltpu.async_copy(
                row0_v.at[pl.ds(r * _REM_B, _REM_B)],
                lin_hbm.at[pl.ds(r * _V + _NFULL * _WIN + _REM_A, _REM_B)],
                osem,
            ).wait()


@functools.partial(
    pl.kernel,
    out_type=jax.ShapeDtypeStruct((_B,), jnp.float32),
    mesh=_mesh,
    scratch_types=[
        pltpu.VMEM((_IPW,), jnp.int32),
        pltpu.VMEM((_IPW,), jnp.float32),
        pltpu.VMEM((_BPW,), jnp.float32),
        pltpu.SemaphoreType.DMA,
    ],
)
def _gather_kernel(idx_hbm, lin_hbm, out_hbm, idx_v, vals_v, out_v, sem):
    wid = lax.axis_index("s") * _NC + lax.axis_index("c")
    pltpu.sync_copy(idx_hbm.at[wid], idx_v)
    pltpu.async_copy(lin_hbm.at[idx_v], vals_v, sem).wait()

    for j in range(_BPW // _L):
        acc = None
        for f in range(_F):
            v = vals_v[pl.ds(f * _BPW + j * _L, _L)]
            acc = v if acc is None else acc + v
        out_v[pl.ds(j * _L, _L)] = 1.0 / (1.0 + jnp.exp(-acc))

    pltpu.sync_copy(out_v, out_hbm.at[pl.ds(wid * _BPW, _BPW)])


def kernel(X, lin_table):
    offs = jnp.arange(_F, dtype=jnp.int32) * _V
    idx = (X + offs[None, :]).T.reshape(_F, _NW, _BPW).transpose(1, 0, 2)
    idx = idx.reshape(_NW, _IPW)
    tail = lin_table[:, _V - _REM_B:].reshape(-1)
    lin = _detile_kernel(lin_table, tail)
    out = _gather_kernel(idx, lin)
    return out.reshape(_B, 1)


# speedup vs baseline: 11.4184x; 1.9094x over previous
import functools

import jax
import jax.numpy as jnp
from jax import lax
from jax.experimental import pallas as pl
from jax.experimental.pallas import tpu as pltpu
from jax.experimental.pallas import tpu_sc as plsc

_F = 26
_V = 1_000_000
_B = 16384
_NC, _NS, _L = 2, 16, 16
_NW = _NC * _NS
_BPW = _B // _NW
_IPW = _BPW * _F

_WIN = 1024
_NFULL = _V // _WIN
_REM = _V - _NFULL * _WIN
_REM_A = (_REM // 128) * 128
_REM_B = _REM - _REM_A

_mesh = plsc.VectorSubcoreMesh(
    core_axis_name="c", subcore_axis_name="s", num_cores=_NC, num_subcores=_NS
)


@functools.partial(
    pl.kernel,
    out_type=jax.ShapeDtypeStruct((_F * _V,), jnp.float32),
    mesh=_mesh,
    scratch_types=[
        pltpu.VMEM((_F, _WIN), jnp.float32),
        pltpu.VMEM((_F, _WIN), jnp.float32),
        pltpu.VMEM((_F * _WIN,), jnp.float32),
        pltpu.SemaphoreType.DMA,
        pltpu.SemaphoreType.DMA,
    ],
)
def _detile_kernel(table_hbm, tail_hbm, lin_hbm, stripe_a, stripe_b, rows_v,
                   isem, osem):
    wid = lax.axis_index("s") * _NC + lax.axis_index("c")
    _ROUNDS = (_NFULL + _NW - 1) // _NW

    def _fetch(w, buf, width=_WIN):
        off = pl.multiple_of(w * _WIN, _WIN)
        return pltpu.async_copy(
            table_hbm.at[:, pl.ds(off, width)],
            buf.at[:, pl.ds(0, width)],
            isem,
        )

    def _wait_fetch(w, buf, width=_WIN):
        off = pl.multiple_of(w * _WIN, _WIN)
        pltpu.make_async_copy(
            table_hbm.at[:, pl.ds(off, width)],
            buf.at[:, pl.ds(0, width)],
            isem,
        ).wait()

    def _extract(w, buf, width=_WIN):
        off = pl.multiple_of(w * _WIN, _WIN)
        descs = []
        for r in range(_F):

            @pl.loop(0, width // _L, unroll=8)
            def _(c):
                rows_v[pl.ds(r * _WIN + c * _L, _L)] = buf[r, pl.ds(c * _L, _L)]

            descs.append(pltpu.async_copy(
                rows_v.at[pl.ds(r * _WIN, width)],
                lin_hbm.at[pl.ds(r * _V + off, width)],
                osem,
            ))
        for d in descs:
            d.wait()

    def _round(k, cur, nxt):
        w = k * _NW + wid

        @pl.when(w < _NFULL)
        def _():
            w2 = w + _NW

            @pl.when(w2 < _NFULL)
            def _():
                _fetch(w2, nxt)

            _wait_fetch(w, cur)
            _extract(w, cur)

    _fetch(wid, stripe_a)

    @pl.loop(0, _ROUNDS + (_ROUNDS % 2), step=2)
    def _(k):
        _round(k, stripe_a, stripe_b)
        _round(k + 1, stripe_b, stripe_a)

    @pl.when(wid == 0)
    def _():
        _fetch(_NFULL, stripe_a, _REM_A).wait()
        _extract(_NFULL, stripe_a, _REM_A)

    @pl.when(wid == 1)
    def _():
        pltpu.sync_copy(tail_hbm, rows_v.at[pl.ds(0, _F * _REM_B)])
        for r in range(_F):
            pltpu.async_copy(
                rows_v.at[pl.ds(r * _REM_B, _REM_B)],
                lin_hbm.at[pl.ds(r * _V + _NFULL * _WIN + _REM_A, _REM_B)],
                osem,
            ).wait()


@functools.partial(
    pl.kernel,
    out_type=jax.ShapeDtypeStruct((_B,), jnp.float32),
    mesh=_mesh,
    scratch_types=[
        pltpu.VMEM((_IPW,), jnp.int32),
        pltpu.VMEM((_IPW,), jnp.float32),
        pltpu.VMEM((_BPW,), jnp.float32),
        pltpu.SemaphoreType.DMA,
    ],
)
def _gather_kernel(idx_hbm, lin_hbm, out_hbm, idx_v, vals_v, out_v, sem):
    wid = lax.axis_index("s") * _NC + lax.axis_index("c")
    pltpu.sync_copy(idx_hbm.at[wid], idx_v)
    pltpu.async_copy(lin_hbm.at[idx_v], vals_v, sem).wait()

    for j in range(_BPW // _L):
        acc = None
        for f in range(_F):
            v = vals_v[pl.ds(f * _BPW + j * _L, _L)]
            acc = v if acc is None else acc + v
        out_v[pl.ds(j * _L, _L)] = 1.0 / (1.0 + jnp.exp(-acc))

    pltpu.sync_copy(out_v, out_hbm.at[pl.ds(wid * _BPW, _BPW)])


def kernel(X, lin_table):
    offs = jnp.arange(_F, dtype=jnp.int32) * _V
    idx = (X + offs[None, :]).T.reshape(_F, _NW, _BPW).transpose(1, 0, 2)
    idx = idx.reshape(_NW, _IPW)
    tail = lin_table[:, _V - _REM_B:].reshape(-1)
    lin = _detile_kernel(lin_table, tail)
    out = _gather_kernel(idx, lin)
    return out.reshape(_B, 1)
